# baseline (speedup 1.0000x reference)
import functools

import jax
import jax.numpy as jnp
from jax import lax
from jax.experimental import pallas as pl
from jax.experimental.pallas import tpu as pltpu
from jax.experimental.pallas import tpu_sc as plsc

_N_SPARSE = 26
_N_DENSE = 13
_VOCAB = 100000
_EMB = 64
_BATCH = 4096
_NC, _NS = 2, 16
_NW = _NC * _NS
_BPW = _BATCH // _NW
_IPW = _N_SPARSE * _BPW
_OUT_D = _N_SPARSE * _EMB + _N_DENSE

_VC = 6400
_NCHUNK = -(-_VOCAB // _VC)

_mesh = plsc.VectorSubcoreMesh(
    core_axis_name="c", subcore_axis_name="s",
    num_cores=_NC, num_subcores=_NS,
)


def _reformat_body(in_ref, out_ref):
    x = in_ref[0]
    eye = jax.lax.broadcasted_iota(jnp.int32, (_EMB, _EMB), 0)
    eye = (eye == jax.lax.broadcasted_iota(jnp.int32, (_EMB, _EMB), 1))
    xt = jax.lax.dot_general(x, eye.astype(jnp.float32),
                             (((0,), (0,)), ((), ())),
                             preferred_element_type=jnp.float32)
    xt = xt.reshape(_VC // 2, 2, _EMB)
    out_ref[0] = jnp.concatenate([xt[:, 0, :], xt[:, 1, :]], axis=1)


_tc_reformat = pl.pallas_call(
    _reformat_body,
    grid=(_N_SPARSE, _NCHUNK),
    in_specs=[pl.BlockSpec((1, _EMB, _VC), lambda i, c: (i, 0, c))],
    out_specs=pl.BlockSpec((1, _VC // 2, 2 * _EMB), lambda i, c: (i, c, 0)),
    out_shape=jax.ShapeDtypeStruct((_N_SPARSE, _VOCAB // 2, 2 * _EMB),
                                   jnp.float32),
    compiler_params=pltpu.CompilerParams(
        dimension_semantics=("parallel", "parallel")),
)


@functools.partial(
    pl.kernel,
    out_type=jax.ShapeDtypeStruct((_BATCH, _OUT_D), jnp.float32),
    mesh=_mesh,
    scratch_types=[
        pltpu.VMEM((_IPW,), jnp.int32),
        pltpu.VMEM((_BPW, _EMB), jnp.float32),
        pltpu.VMEM((_BPW, _N_DENSE), jnp.float32),
        pltpu.SemaphoreType.DMA,
    ],
    compiler_params=pltpu.CompilerParams(use_tc_tiling_on_sc=False),
)
def _sc_embed(tab_hbm, idx_hbm, dense_hbm, out_hbm, idx_v, rows_v, dense_v, sem):
    wid = lax.axis_index("s") * _NC + lax.axis_index("c")
    base = wid * _BPW

    pltpu.sync_copy(idx_hbm.at[pl.ds(wid * _IPW, _IPW)], idx_v)

    pltpu.sync_copy(dense_hbm.at[pl.ds(base, _BPW), :], dense_v)
    pltpu.sync_copy(dense_v,
                    out_hbm.at[pl.ds(base, _BPW),
                               pl.ds(_N_SPARSE * _EMB, _N_DENSE)])

    def body(i, carry):
        pltpu.async_copy(tab_hbm.at[idx_v.at[pl.ds(i * _BPW, _BPW)]],
                         rows_v, sem).wait()
        pltpu.sync_copy(rows_v,
                        out_hbm.at[pl.ds(base, _BPW), pl.ds(i * _EMB, _EMB)])
        return carry

    lax.fori_loop(0, _N_SPARSE, body, 0)


def kernel(inputs, tables):
    sp = inputs[:, :_N_SPARSE].astype(jnp.int32)
    gidx = (jnp.transpose(sp)
            + (jnp.arange(_N_SPARSE, dtype=jnp.int32) * _VOCAB)[:, None])
    idx1d = gidx.reshape(_N_SPARSE, _NW, _BPW).transpose(1, 0, 2).reshape(-1)
    tab_t = tables.transpose(0, 2, 1)
    tab_pairs = _tc_reformat(tab_t)
    tab_flat = tab_pairs.reshape(_N_SPARSE * _VOCAB, _EMB)
    dense = inputs[:, _N_SPARSE:]
    return _sc_embed(tab_flat, idx1d, dense)


# speedup vs baseline: 1.0854x; 1.0854x over previous
import functools

import jax
import jax.numpy as jnp
from jax import lax
from jax.experimental import pallas as pl
from jax.experimental.pallas import tpu as pltpu
from jax.experimental.pallas import tpu_sc as plsc

_N_SPARSE = 26
_N_DENSE = 13
_VOCAB = 100000
_EMB = 64
_BATCH = 4096
_NC, _NS = 2, 16
_NW = _NC * _NS
_BPW = _BATCH // _NW
_IPW = _N_SPARSE * _BPW
_OUT_D = _N_SPARSE * _EMB + _N_DENSE

_VC = 6400
_NCHUNK = -(-_VOCAB // _VC)

_mesh = plsc.VectorSubcoreMesh(
    core_axis_name="c", subcore_axis_name="s",
    num_cores=_NC, num_subcores=_NS,
)


def _reformat_body(in_ref, out_ref):
    x = in_ref[0]
    xt = x.T.reshape(_VC // 2, 2, _EMB)
    out_ref[0] = jnp.concatenate([xt[:, 0, :], xt[:, 1, :]], axis=1)


_tc_reformat = pl.pallas_call(
    _reformat_body,
    grid=(_N_SPARSE, _NCHUNK),
    in_specs=[pl.BlockSpec((1, _EMB, _VC), lambda i, c: (i, 0, c))],
    out_specs=pl.BlockSpec((1, _VC // 2, 2 * _EMB), lambda i, c: (i, c, 0)),
    out_shape=jax.ShapeDtypeStruct((_N_SPARSE, _VOCAB // 2, 2 * _EMB),
                                   jnp.float32),
    compiler_params=pltpu.CompilerParams(
        dimension_semantics=("parallel", "parallel")),
)


@functools.partial(
    pl.kernel,
    out_type=jax.ShapeDtypeStruct((_BATCH, _OUT_D), jnp.float32),
    mesh=_mesh,
    scratch_types=[
        pltpu.VMEM((_IPW,), jnp.int32),
        pltpu.VMEM((_BPW, _EMB), jnp.float32),
        pltpu.VMEM((_BPW, _N_DENSE), jnp.float32),
        pltpu.SemaphoreType.DMA,
    ],
    compiler_params=pltpu.CompilerParams(use_tc_tiling_on_sc=False),
)
def _sc_embed(tab_hbm, idx_hbm, dense_hbm, out_hbm, idx_v, rows_v, dense_v, sem):
    wid = lax.axis_index("s") * _NC + lax.axis_index("c")
    base = wid * _BPW

    pltpu.sync_copy(idx_hbm.at[pl.ds(wid * _IPW, _IPW)], idx_v)

    pltpu.sync_copy(dense_hbm.at[pl.ds(base, _BPW), :], dense_v)
    pltpu.sync_copy(dense_v,
                    out_hbm.at[pl.ds(base, _BPW),
                               pl.ds(_N_SPARSE * _EMB, _N_DENSE)])

    def body(i, carry):
        pltpu.async_copy(tab_hbm.at[idx_v.at[pl.ds(i * _BPW, _BPW)]],
                         rows_v, sem).wait()
        pltpu.sync_copy(rows_v,
                        out_hbm.at[pl.ds(base, _BPW), pl.ds(i * _EMB, _EMB)])
        return carry

    lax.fori_loop(0, _N_SPARSE, body, 0)


def kernel(inputs, tables):
    sp = inputs[:, :_N_SPARSE].astype(jnp.int32)
    gidx = (jnp.transpose(sp)
            + (jnp.arange(_N_SPARSE, dtype=jnp.int32) * _VOCAB)[:, None])
    idx1d = gidx.reshape(_N_SPARSE, _NW, _BPW).transpose(1, 0, 2).reshape(-1)
    tab_t = tables.transpose(0, 2, 1)
    tab_pairs = _tc_reformat(tab_t)
    tab_flat = tab_pairs.reshape(_N_SPARSE * _VOCAB, _EMB)
    dense = inputs[:, _N_SPARSE:]
    return _sc_embed(tab_flat, idx1d, dense)
